# baseline (speedup 1.0000x reference)
import functools

import jax
import jax.numpy as jnp
import numpy as np
from jax import lax
from jax.experimental import pallas as pl
from jax.experimental.pallas import tpu as pltpu
from jax.experimental.pallas import tpu_sc as plsc

B = 128
V = 100000
NC = 2
NS = 16
NW = NC * NS
ROWS_PER_W = B // NW
CHUNK = 10000
NCHUNK = V // CHUNK
L = 16
UNROLL = 5
GROUPS = CHUNK // (L * UNROLL)

_G_CACHE = []


def _gumbel_const():
    if not _G_CACHE:
        _G_CACHE.append(
            jax.random.gumbel(jax.random.key(42), (B, V), jnp.float32))
    return _G_CACHE[0]

_NEG_INF = np.float32(-np.inf)
_BIG_I32 = np.int32(2**30)


def _merge(va, ia, vb, ib):
    take_b = (vb > va) | ((vb == va) & (ib < ia))
    return jnp.where(take_b, vb, va), jnp.where(take_b, ib, ia)


def _sc_body(logits_hbm, g_hbm, out_hbm, lbufs, gbufs, res_vm, sems):
    wid = lax.axis_index("s") * NC + lax.axis_index("c")
    lane = lax.iota(jnp.int32, L)
    res = jnp.zeros((L,), jnp.int32)

    for rl in range(ROWS_PER_W):
        row = wid * ROWS_PER_W + rl

        def dma_pair(ch, slot):
            cols = pl.ds(ch * CHUNK, CHUNK)
            cl = pltpu.make_async_copy(logits_hbm.at[row, cols], lbufs[slot],
                                       sems[2 * slot])
            cg = pltpu.make_async_copy(g_hbm.at[row, cols], gbufs[slot],
                                       sems[2 * slot + 1])
            cl.start()
            cg.start()
            return cl, cg

        bests = [jnp.full((L,), _NEG_INF, jnp.float32) for _ in range(UNROLL)]
        bidxs = [jnp.zeros((L,), jnp.int32) for _ in range(UNROLL)]

        pending = dma_pair(0, 0)
        for ch in range(NCHUNK):
            pending[0].wait()
            pending[1].wait()
            slot = ch % 2
            if ch + 1 < NCHUNK:
                pending = dma_pair(ch + 1, (ch + 1) % 2)
            lbuf, gbuf = lbufs[slot], gbufs[slot]
            base0 = ch * CHUNK

            def step(i, carry):
                accs = list(carry)
                off = i * (L * UNROLL)
                for j in range(UNROLL):
                    lv = lbuf[pl.ds(off + j * L, L)]
                    gv = gbuf[pl.ds(off + j * L, L)]
                    v = lv + gv
                    cols_v = (base0 + j * L) + off + lane
                    m = v > accs[2 * j]
                    accs[2 * j] = jnp.where(m, v, accs[2 * j])
                    accs[2 * j + 1] = jnp.where(m, cols_v, accs[2 * j + 1])
                return tuple(accs)

            flat = []
            for j in range(UNROLL):
                flat += [bests[j], bidxs[j]]
            flat = lax.fori_loop(0, GROUPS, step, tuple(flat))
            bests = [flat[2 * j] for j in range(UNROLL)]
            bidxs = [flat[2 * j + 1] for j in range(UNROLL)]

        bv, bi = bests[0], bidxs[0]
        for j in range(1, UNROLL):
            bv, bi = _merge(bv, bi, bests[j], bidxs[j])
        for s in (8, 4, 2, 1):
            perm = jnp.bitwise_xor(lane, np.int32(s))
            pv = jnp.take_along_axis(bv, perm, axis=0)
            pi = jnp.take_along_axis(bi, perm, axis=0)
            bv, bi = _merge(bv, bi, pv, pi)
        res = jnp.where(lane == rl, bi, res)

    res_vm[...] = res
    pltpu.sync_copy(res_vm, out_hbm.at[wid])


@functools.partial(jax.jit, donate_argnums=())
def _sc_argmax(logits, gumbel):
    mesh = plsc.VectorSubcoreMesh(core_axis_name="c", subcore_axis_name="s")
    kfn = pl.kernel(
        _sc_body,
        out_type=jax.ShapeDtypeStruct((NW, L), jnp.int32),
        mesh=mesh,
        scratch_types=[
            [pltpu.VMEM((CHUNK,), jnp.float32) for _ in range(2)],
            [pltpu.VMEM((CHUNK,), jnp.float32) for _ in range(2)],
            pltpu.VMEM((L,), jnp.int32),
            [pltpu.SemaphoreType.DMA for _ in range(4)],
        ],
        compiler_params=pltpu.CompilerParams(use_tc_tiling_on_sc=False),
        name="gumbel_argmax_sc",
    )
    out = kfn(logits, gumbel)
    return out[:, :ROWS_PER_W].reshape(B)


def kernel(logits):
    return _sc_argmax(logits, _gumbel_const())


# speedup vs baseline: 1.3673x; 1.3673x over previous
import functools

import jax
import jax.numpy as jnp
import numpy as np
from jax import lax
from jax.experimental import pallas as pl
from jax.experimental.pallas import tpu as pltpu
from jax.experimental.pallas import tpu_sc as plsc

B = 128
V = 100000

TC_CHUNK = 2048
V_TC = V
V_SC = V - V_TC

NC = 2
NS = 16
NW = NC * NS
ROWS_PER_W = B // NW
L = 16
UNROLL = 5

_G_CACHE = []


def _gumbel_const():
    if not _G_CACHE:
        _G_CACHE.append(
            jax.random.gumbel(jax.random.key(42), (B, V), jnp.float32))
    return _G_CACHE[0]

_NEG_INF = np.float32(-np.inf)
_BIG_I32 = np.int32(2**30)


def _merge(va, ia, vb, ib):
    take_b = (vb > va) | ((vb == va) & (ib < ia))
    return jnp.where(take_b, vb, va), jnp.where(take_b, ib, ia)


def _tc_body(l_ref, g_ref, ov_ref, oi_ref, bv_ref, bi_ref):
    i = pl.program_id(0)
    v = l_ref[...] + g_ref[...]
    cols = lax.broadcasted_iota(jnp.int32, v.shape, 1)
    v = jnp.where(i * TC_CHUNK + cols < V_TC, v, _NEG_INF)
    m = jnp.max(v, axis=1)
    a = jnp.min(jnp.where(v == m[:, None], cols, _BIG_I32), axis=1)
    idx = i * TC_CHUNK + a

    @pl.when(i == 0)
    def _():
        bv_ref[...] = m
        bi_ref[...] = idx

    @pl.when(i > 0)
    def _():
        better = m > bv_ref[...]
        bv_ref[...] = jnp.where(better, m, bv_ref[...])
        bi_ref[...] = jnp.where(better, idx, bi_ref[...])

    @pl.when(i == pl.num_programs(0) - 1)
    def _():
        ov_ref[...] = bv_ref[...]
        oi_ref[...] = bi_ref[...]


def _tc_argmax(logits, gumbel):
    nblk = pl.cdiv(V_TC, TC_CHUNK)
    return pl.pallas_call(
        _tc_body,
        grid=(nblk,),
        in_specs=[
            pl.BlockSpec((B, TC_CHUNK), lambda i: (0, i)),
            pl.BlockSpec((B, TC_CHUNK), lambda i: (0, i)),
        ],
        out_specs=[
            pl.BlockSpec((B,), lambda i: (0,)),
            pl.BlockSpec((B,), lambda i: (0,)),
        ],
        out_shape=[
            jax.ShapeDtypeStruct((B,), jnp.float32),
            jax.ShapeDtypeStruct((B,), jnp.int32),
        ],
        scratch_shapes=[
            pltpu.VMEM((B,), jnp.float32),
            pltpu.VMEM((B,), jnp.int32),
        ],
        name="gumbel_argmax_tc",
    )(logits, gumbel)


def _sc_body(logits_hbm, g_hbm, oi_hbm, ov_hbm, lbufs, gbufs, res_i, res_v,
             sems, *, chunk, nchunk):
    wid = lax.axis_index("s") * NC + lax.axis_index("c")
    lane = lax.iota(jnp.int32, L)
    acc_i = jnp.zeros((L,), jnp.int32)
    acc_v = jnp.full((L,), _NEG_INF, jnp.float32)
    groups = chunk // (L * UNROLL)

    for rl in range(ROWS_PER_W):
        row = wid * ROWS_PER_W + rl

        def dma_pair(ch, slot):
            cols = pl.ds(ch * chunk, chunk)
            cl = pltpu.make_async_copy(logits_hbm.at[row, cols], lbufs[slot],
                                       sems[2 * slot])
            cg = pltpu.make_async_copy(g_hbm.at[row, cols], gbufs[slot],
                                       sems[2 * slot + 1])
            cl.start()
            cg.start()
            return cl, cg

        bests = [jnp.full((L,), _NEG_INF, jnp.float32) for _ in range(UNROLL)]
        bidxs = [jnp.zeros((L,), jnp.int32) for _ in range(UNROLL)]

        pending = dma_pair(0, 0)
        for ch in range(nchunk):
            pending[0].wait()
            pending[1].wait()
            slot = ch % 2
            if ch + 1 < nchunk:
                pending = dma_pair(ch + 1, (ch + 1) % 2)
            lbuf, gbuf = lbufs[slot], gbufs[slot]
            base0 = ch * chunk

            def step(i, carry):
                accs = list(carry)
                off = i * (L * UNROLL)
                for j in range(UNROLL):
                    lv = lbuf[pl.ds(off + j * L, L)]
                    gv = gbuf[pl.ds(off + j * L, L)]
                    v = lv + gv
                    cols_v = (base0 + j * L) + off + lane
                    m = v > accs[2 * j]
                    accs[2 * j] = jnp.where(m, v, accs[2 * j])
                    accs[2 * j + 1] = jnp.where(m, cols_v, accs[2 * j + 1])
                return tuple(accs)

            flat = []
            for j in range(UNROLL):
                flat += [bests[j], bidxs[j]]
            flat = lax.fori_loop(0, groups, step, tuple(flat))
            bests = [flat[2 * j] for j in range(UNROLL)]
            bidxs = [flat[2 * j + 1] for j in range(UNROLL)]

        bv, bi = bests[0], bidxs[0]
        for j in range(1, UNROLL):
            bv, bi = _merge(bv, bi, bests[j], bidxs[j])
        for s in (8, 4, 2, 1):
            perm = jnp.bitwise_xor(lane, np.int32(s))
            pv = jnp.take_along_axis(bv, perm, axis=0)
            pi = jnp.take_along_axis(bi, perm, axis=0)
            bv, bi = _merge(bv, bi, pv, pi)
        acc_i = jnp.where(lane == rl, bi, acc_i)
        acc_v = jnp.where(lane == rl, bv, acc_v)

    res_i[...] = acc_i
    res_v[...] = acc_v
    pltpu.sync_copy(res_i, oi_hbm.at[wid])
    pltpu.sync_copy(res_v, ov_hbm.at[wid])


def _sc_argmax(logits_sc, gumbel_sc, chunk):
    nchunk = V_SC // chunk
    mesh = plsc.VectorSubcoreMesh(core_axis_name="c", subcore_axis_name="s")
    kfn = pl.kernel(
        functools.partial(_sc_body, chunk=chunk, nchunk=nchunk),
        out_type=[
            jax.ShapeDtypeStruct((NW, L), jnp.int32),
            jax.ShapeDtypeStruct((NW, L), jnp.float32),
        ],
        mesh=mesh,
        scratch_types=[
            [pltpu.VMEM((chunk,), jnp.float32) for _ in range(2)],
            [pltpu.VMEM((chunk,), jnp.float32) for _ in range(2)],
            pltpu.VMEM((L,), jnp.int32),
            pltpu.VMEM((L,), jnp.float32),
            [pltpu.SemaphoreType.DMA for _ in range(4)],
        ],
        compiler_params=pltpu.CompilerParams(use_tc_tiling_on_sc=False),
        name="gumbel_argmax_sc",
    )
    oi, ov = kfn(logits_sc, gumbel_sc)
    return (oi[:, :ROWS_PER_W].reshape(B), ov[:, :ROWS_PER_W].reshape(B))


@jax.jit
def _sample(logits):
    g = _gumbel_const()
    if V_SC == 0:
        _, ti = _tc_argmax(logits, g)
        return ti
    if V_TC == 0:
        si, _ = _sc_argmax(logits, g, chunk=V_SC // 10 if V_SC >= 10000 else V_SC)
        return si
    tv, ti = _tc_argmax(logits[:, :V_TC], g[:, :V_TC])
    chunk = V_SC // 2 if V_SC > 10000 else V_SC
    si, sv = _sc_argmax(logits[:, V_TC:], g[:, V_TC:], chunk=chunk)
    si = si + V_TC
    return jnp.where(sv > tv, si, ti)


def kernel(logits):
    return _sample(logits)


# speedup vs baseline: 1.4836x; 1.0850x over previous
import functools

import jax
import jax.numpy as jnp
import numpy as np
from jax import lax
from jax.experimental import pallas as pl
from jax.experimental.pallas import tpu as pltpu
from jax.experimental.pallas import tpu_sc as plsc

B = 128
V = 100000

TC_CHUNK = 8192
V_TC = V
V_SC = V - V_TC

NC = 2
NS = 16
NW = NC * NS
ROWS_PER_W = B // NW
L = 16
UNROLL = 5

_G_CACHE = []


def _gumbel_const():
    if not _G_CACHE:
        _G_CACHE.append(
            jax.random.gumbel(jax.random.key(42), (B, V), jnp.float32))
    return _G_CACHE[0]

_NEG_INF = np.float32(-np.inf)
_BIG_I32 = np.int32(2**30)


def _merge(va, ia, vb, ib):
    take_b = (vb > va) | ((vb == va) & (ib < ia))
    return jnp.where(take_b, vb, va), jnp.where(take_b, ib, ia)


def _tc_body(l_ref, g_ref, ov_ref, oi_ref, bv_ref, bi_ref):
    i = pl.program_id(0)
    v = l_ref[...] + g_ref[...]
    cols = lax.broadcasted_iota(jnp.int32, v.shape, 1)
    v = jnp.where(i * TC_CHUNK + cols < V_TC, v, _NEG_INF)
    m = jnp.max(v, axis=1)
    a = jnp.min(jnp.where(v == m[:, None], cols, _BIG_I32), axis=1)
    idx = i * TC_CHUNK + a

    @pl.when(i == 0)
    def _():
        bv_ref[...] = m
        bi_ref[...] = idx

    @pl.when(i > 0)
    def _():
        better = m > bv_ref[...]
        bv_ref[...] = jnp.where(better, m, bv_ref[...])
        bi_ref[...] = jnp.where(better, idx, bi_ref[...])

    @pl.when(i == pl.num_programs(0) - 1)
    def _():
        ov_ref[...] = bv_ref[...]
        oi_ref[...] = bi_ref[...]


def _tc_argmax(logits, gumbel):
    nblk = pl.cdiv(V_TC, TC_CHUNK)
    return pl.pallas_call(
        _tc_body,
        grid=(nblk,),
        in_specs=[
            pl.BlockSpec((B, TC_CHUNK), lambda i: (0, i)),
            pl.BlockSpec((B, TC_CHUNK), lambda i: (0, i)),
        ],
        out_specs=[
            pl.BlockSpec((B,), lambda i: (0,)),
            pl.BlockSpec((B,), lambda i: (0,)),
        ],
        out_shape=[
            jax.ShapeDtypeStruct((B,), jnp.float32),
            jax.ShapeDtypeStruct((B,), jnp.int32),
        ],
        scratch_shapes=[
            pltpu.VMEM((B,), jnp.float32),
            pltpu.VMEM((B,), jnp.int32),
        ],
        name="gumbel_argmax_tc",
    )(logits, gumbel)


def _sc_body(logits_hbm, g_hbm, oi_hbm, ov_hbm, lbufs, gbufs, res_i, res_v,
             sems, *, chunk, nchunk):
    wid = lax.axis_index("s") * NC + lax.axis_index("c")
    lane = lax.iota(jnp.int32, L)
    acc_i = jnp.zeros((L,), jnp.int32)
    acc_v = jnp.full((L,), _NEG_INF, jnp.float32)
    groups = chunk // (L * UNROLL)

    for rl in range(ROWS_PER_W):
        row = wid * ROWS_PER_W + rl

        def dma_pair(ch, slot):
            cols = pl.ds(ch * chunk, chunk)
            cl = pltpu.make_async_copy(logits_hbm.at[row, cols], lbufs[slot],
                                       sems[2 * slot])
            cg = pltpu.make_async_copy(g_hbm.at[row, cols], gbufs[slot],
                                       sems[2 * slot + 1])
            cl.start()
            cg.start()
            return cl, cg

        bests = [jnp.full((L,), _NEG_INF, jnp.float32) for _ in range(UNROLL)]
        bidxs = [jnp.zeros((L,), jnp.int32) for _ in range(UNROLL)]

        pending = dma_pair(0, 0)
        for ch in range(nchunk):
            pending[0].wait()
            pending[1].wait()
            slot = ch % 2
            if ch + 1 < nchunk:
                pending = dma_pair(ch + 1, (ch + 1) % 2)
            lbuf, gbuf = lbufs[slot], gbufs[slot]
            base0 = ch * chunk

            def step(i, carry):
                accs = list(carry)
                off = i * (L * UNROLL)
                for j in range(UNROLL):
                    lv = lbuf[pl.ds(off + j * L, L)]
                    gv = gbuf[pl.ds(off + j * L, L)]
                    v = lv + gv
                    cols_v = (base0 + j * L) + off + lane
                    m = v > accs[2 * j]
                    accs[2 * j] = jnp.where(m, v, accs[2 * j])
                    accs[2 * j + 1] = jnp.where(m, cols_v, accs[2 * j + 1])
                return tuple(accs)

            flat = []
            for j in range(UNROLL):
                flat += [bests[j], bidxs[j]]
            flat = lax.fori_loop(0, groups, step, tuple(flat))
            bests = [flat[2 * j] for j in range(UNROLL)]
            bidxs = [flat[2 * j + 1] for j in range(UNROLL)]

        bv, bi = bests[0], bidxs[0]
        for j in range(1, UNROLL):
            bv, bi = _merge(bv, bi, bests[j], bidxs[j])
        for s in (8, 4, 2, 1):
            perm = jnp.bitwise_xor(lane, np.int32(s))
            pv = jnp.take_along_axis(bv, perm, axis=0)
            pi = jnp.take_along_axis(bi, perm, axis=0)
            bv, bi = _merge(bv, bi, pv, pi)
        acc_i = jnp.where(lane == rl, bi, acc_i)
        acc_v = jnp.where(lane == rl, bv, acc_v)

    res_i[...] = acc_i
    res_v[...] = acc_v
    pltpu.sync_copy(res_i, oi_hbm.at[wid])
    pltpu.sync_copy(res_v, ov_hbm.at[wid])


def _sc_argmax(logits_sc, gumbel_sc, chunk):
    nchunk = V_SC // chunk
    mesh = plsc.VectorSubcoreMesh(core_axis_name="c", subcore_axis_name="s")
    kfn = pl.kernel(
        functools.partial(_sc_body, chunk=chunk, nchunk=nchunk),
        out_type=[
            jax.ShapeDtypeStruct((NW, L), jnp.int32),
            jax.ShapeDtypeStruct((NW, L), jnp.float32),
        ],
        mesh=mesh,
        scratch_types=[
            [pltpu.VMEM((chunk,), jnp.float32) for _ in range(2)],
            [pltpu.VMEM((chunk,), jnp.float32) for _ in range(2)],
            pltpu.VMEM((L,), jnp.int32),
            pltpu.VMEM((L,), jnp.float32),
            [pltpu.SemaphoreType.DMA for _ in range(4)],
        ],
        compiler_params=pltpu.CompilerParams(use_tc_tiling_on_sc=False),
        name="gumbel_argmax_sc",
    )
    oi, ov = kfn(logits_sc, gumbel_sc)
    return (oi[:, :ROWS_PER_W].reshape(B), ov[:, :ROWS_PER_W].reshape(B))


@jax.jit
def _sample(logits):
    g = _gumbel_const()
    if V_SC == 0:
        _, ti = _tc_argmax(logits, g)
        return ti
    if V_TC == 0:
        si, _ = _sc_argmax(logits, g, chunk=V_SC // 10 if V_SC >= 10000 else V_SC)
        return si
    tv, ti = _tc_argmax(logits[:, :V_TC], g[:, :V_TC])
    chunk = V_SC // 2 if V_SC > 10000 else V_SC
    si, sv = _sc_argmax(logits[:, V_TC:], g[:, V_TC:], chunk=chunk)
    si = si + V_TC
    return jnp.where(sv > tv, si, ti)


def kernel(logits):
    return _sample(logits)
